# baseline (speedup 1.0000x reference)
import functools

import jax
import jax.numpy as jnp
import numpy as np
from jax import lax
from jax.experimental import pallas as pl
from jax.experimental.pallas import tpu as pltpu
from jax.experimental.pallas import tpu_sc as plsc

_B, _N, _C = 32, 8192, 64
_NPOINTS = 2048
_SROWS = 2
_NSTRIPE = _C // _SROWS

_IDX_CONST = None


def _index_consts() -> np.ndarray:
    global _IDX_CONST
    if _IDX_CONST is None:
        with jax.ensure_compile_time_eval():
            idx = jax.random.randint(
                jax.random.key(42), (_B, _NPOINTS), 0, _N, dtype=jnp.int32)
            _IDX_CONST = np.asarray(idx)
    return _IDX_CONST


def _sampler_body(table_hbm, idx_hbm, out_hbm,
                  idx_v, inbuf, outbuf, isem0, isem1, isem2, osem0, osem1):
    isems, osems = (isem0, isem1, isem2), (osem0, osem1)
    b = lax.axis_index("s") * 2 + lax.axis_index("c")
    row0 = b * _C
    rsplats = [jnp.full((16,), r, jnp.int32) for r in range(_SROWS)]

    def start_in(s):
        ph = s % 3
        return pltpu.async_copy(
            table_hbm.at[pl.ds(row0 + s * _SROWS, _SROWS)],
            inbuf.at[ph], isems[ph])

    def gather(s):
        ph = s % 3
        src = inbuf.at[ph]
        dst = outbuf.at[s % 2]

        def body(i, carry):
            base = i * 128
            idxvs = [idx_v[pl.ds(base + u * 16, 16)] for u in range(8)]
            vals = [plsc.load_gather(src, [rsplats[r], idxvs[u]])
                    for u in range(8) for r in range(_SROWS)]
            for u in range(8):
                for r in range(_SROWS):
                    dst[r, pl.ds(base + u * 16, 16)] = vals[u * _SROWS + r]
            return carry

        lax.fori_loop(0, _NPOINTS // 128, body, 0)

    ih = {}
    for t in range(3):
        ih[t] = start_in(t)
    pltpu.sync_copy(idx_hbm.at[b], idx_v)
    oh = {}
    for s in range(_NSTRIPE):
        ih[s].wait()
        if s >= 2:
            oh[s - 2].wait()
        gather(s)
        if s + 3 < _NSTRIPE:
            ih[s + 3] = start_in(s + 3)
        oh[s] = pltpu.async_copy(
            outbuf.at[s % 2],
            out_hbm.at[pl.ds(row0 + s * _SROWS, _SROWS)], osems[s % 2])
    oh[_NSTRIPE - 2].wait()
    oh[_NSTRIPE - 1].wait()


@functools.partial(jax.jit, static_argnames=())
def _sampler(table, idx):
    mesh = plsc.VectorSubcoreMesh(core_axis_name="c", subcore_axis_name="s")
    call = pl.kernel(
        _sampler_body,
        out_type=jax.ShapeDtypeStruct((_B * _C, _NPOINTS), jnp.float32),
        mesh=mesh,
        scratch_types=[
            pltpu.VMEM((_NPOINTS,), jnp.int32),
            pltpu.VMEM((3, _SROWS, _N), jnp.float32),
            pltpu.VMEM((2, _SROWS, _NPOINTS), jnp.float32),
            pltpu.SemaphoreType.DMA,
            pltpu.SemaphoreType.DMA,
            pltpu.SemaphoreType.DMA,
            pltpu.SemaphoreType.DMA,
            pltpu.SemaphoreType.DMA,
        ],
        compiler_params=pltpu.CompilerParams(needs_layout_passes=False),
    )
    return call(table, idx)


def kernel(inputs):
    table = jnp.transpose(inputs, (0, 2, 1)).reshape(_B * _C, _N)
    idx = jnp.asarray(_index_consts())
    out = _sampler(table, idx)
    return jnp.transpose(out.reshape(_B, _C, _NPOINTS), (0, 2, 1))


# speedup vs baseline: 1.0715x; 1.0715x over previous
import functools

import jax
import jax.numpy as jnp
import numpy as np
from jax import lax
from jax.experimental import pallas as pl
from jax.experimental.pallas import tpu as pltpu
from jax.experimental.pallas import tpu_sc as plsc

_B, _N, _C = 32, 8192, 64
_NPOINTS = 2048
_SROWS = 4
_NSTRIPE = _C // _SROWS

_IDX_CONST = None


def _index_consts() -> np.ndarray:
    global _IDX_CONST
    if _IDX_CONST is None:
        with jax.ensure_compile_time_eval():
            idx = jax.random.randint(
                jax.random.key(42), (_B, _NPOINTS), 0, _N, dtype=jnp.int32)
            _IDX_CONST = np.asarray(idx)
    return _IDX_CONST


def _sampler_body(table_hbm, idx_hbm, out_hbm,
                  idx_v, inbuf, outbuf, isem0, isem1, isem2, osem0, osem1):
    isems, osems = (isem0, isem1, isem2), (osem0, osem1)
    b = lax.axis_index("s") * 2 + lax.axis_index("c")
    row0 = b * _C
    rsplats = [jnp.full((16,), r, jnp.int32) for r in range(_SROWS)]

    def start_in(s):
        ph = s % 3
        return pltpu.async_copy(
            table_hbm.at[pl.ds(row0 + s * _SROWS, _SROWS)],
            inbuf.at[ph], isems[ph])

    def gather(s):
        ph = s % 3
        src = inbuf.at[ph]
        dst = outbuf.at[s % 2]

        def body(i, carry):
            base = i * 128
            idxvs = [idx_v[pl.ds(base + u * 16, 16)] for u in range(8)]
            vals = [plsc.load_gather(src, [rsplats[r], idxvs[u]])
                    for u in range(8) for r in range(_SROWS)]
            for u in range(8):
                for r in range(_SROWS):
                    dst[r, pl.ds(base + u * 16, 16)] = vals[u * _SROWS + r]
            return carry

        lax.fori_loop(0, _NPOINTS // 128, body, 0)

    ih = {}
    for t in range(3):
        ih[t] = start_in(t)
    pltpu.sync_copy(idx_hbm.at[b], idx_v)
    oh = {}
    for s in range(_NSTRIPE):
        ih[s].wait()
        if s >= 2:
            oh[s - 2].wait()
        gather(s)
        if s + 3 < _NSTRIPE:
            ih[s + 3] = start_in(s + 3)
        oh[s] = pltpu.async_copy(
            outbuf.at[s % 2],
            out_hbm.at[pl.ds(row0 + s * _SROWS, _SROWS)], osems[s % 2])
    oh[_NSTRIPE - 2].wait()
    oh[_NSTRIPE - 1].wait()


@functools.partial(jax.jit, static_argnames=())
def _sampler(table, idx):
    mesh = plsc.VectorSubcoreMesh(core_axis_name="c", subcore_axis_name="s")
    call = pl.kernel(
        _sampler_body,
        out_type=jax.ShapeDtypeStruct((_B * _C, _NPOINTS), jnp.float32),
        mesh=mesh,
        scratch_types=[
            pltpu.VMEM((_NPOINTS,), jnp.int32),
            pltpu.VMEM((3, _SROWS, _N), jnp.float32),
            pltpu.VMEM((2, _SROWS, _NPOINTS), jnp.float32),
            pltpu.SemaphoreType.DMA,
            pltpu.SemaphoreType.DMA,
            pltpu.SemaphoreType.DMA,
            pltpu.SemaphoreType.DMA,
            pltpu.SemaphoreType.DMA,
        ],
        compiler_params=pltpu.CompilerParams(needs_layout_passes=False),
    )
    return call(table, idx)


def kernel(inputs):
    table = jnp.transpose(inputs, (0, 2, 1)).reshape(_B * _C, _N)
    idx = jnp.asarray(_index_consts())
    out = _sampler(table, idx)
    return jnp.transpose(out.reshape(_B, _C, _NPOINTS), (0, 2, 1))
